# Initial kernel scaffold; baseline (speedup 1.0000x reference)
import jax
import jax.numpy as jnp
from jax.experimental import pallas as pl


def kernel(inputs, W, b):
    raise NotImplementedError("write your pallas kernel here")


# speedup vs baseline: 3.9393x; 3.9393x over previous
import numpy as np
import jax
import jax.numpy as jnp
from jax.experimental import pallas as pl
from jax.experimental.pallas import tpu as pltpu

_B, _K, _N, _M = 256, 8, 64, 32
_P = _N * (_N - 1) // 2
_R = _B * _K

_ROWS, _COLS = np.triu_indices(_N, k=1)
_OFF = np.concatenate([[0], np.cumsum(np.arange(_N - 1, 0, -1))]).astype(np.int64)

_RT_NP = np.zeros((_N, _P), np.float32)
_RT_NP[_ROWS, np.arange(_P)] += 1.0
_RT_NP[_COLS, np.arange(_P)] += 1.0


def _logic_kernel(x_ref, w_ref, b_ref, rt_ref, out_ref, w2_ref):
    wa = w_ref[0:_P, :]
    wo = w_ref[_P:2 * _P, :]
    wx = w_ref[2 * _P:3 * _P, :]
    a = wa - wo - 2.0 * wx
    c = wo + wx

    w2_ref[:, :] = jnp.zeros((_N * _N, _M), jnp.float32)
    for i in range(_N - 1):
        cnt = _N - 1 - i
        w2_ref[i * _N + i + 1:i * _N + _N, :] = a[int(_OFF[i]):int(_OFF[i]) + cnt, :]

    clin = jnp.dot(rt_ref[:, :], c, preferred_element_type=jnp.float32)

    x = x_ref[:, :]
    acc0 = jnp.dot(x, clin, preferred_element_type=jnp.float32) + b_ref[:, :]
    acc1 = jnp.zeros((_R, _M), jnp.float32)
    acc2 = jnp.zeros((_R, _M), jnp.float32)
    acc3 = jnp.zeros((_R, _M), jnp.float32)
    accs = [acc0, acc1, acc2, acc3]
    for g in range(_N // 2):
        lhs = jnp.concatenate(
            [x * x[:, 2 * g:2 * g + 1], x * x[:, 2 * g + 1:2 * g + 2]], axis=1)
        wblk = w2_ref[128 * g:128 * (g + 1), :]
        accs[g % 4] = accs[g % 4] + jnp.dot(
            lhs, wblk, preferred_element_type=jnp.float32)
    out_ref[:, :] = (accs[0] + accs[1]) + (accs[2] + accs[3])


def kernel(inputs, W, b):
    x2d = inputs.reshape(_R, _N)
    b2d = b.reshape(1, _M)
    rt = jnp.asarray(_RT_NP)
    out = pl.pallas_call(
        _logic_kernel,
        out_shape=jax.ShapeDtypeStruct((_R, _M), jnp.float32),
        scratch_shapes=[pltpu.VMEM((_N * _N, _M), jnp.float32)],
    )(x2d, W, b2d, rt)
    return out.reshape(_B, _K, _M)


# speedup vs baseline: 3.9555x; 1.0041x over previous
import numpy as np
import jax
import jax.numpy as jnp
from jax.experimental import pallas as pl
from jax.experimental.pallas import tpu as pltpu

_B, _K, _N, _M = 256, 8, 64, 32
_P = _N * (_N - 1) // 2
_R = _B * _K

_ROWS, _COLS = np.triu_indices(_N, k=1)
_OFF = np.concatenate([[0], np.cumsum(np.arange(_N - 1, 0, -1))]).astype(np.int64)

_RT_NP = np.zeros((_N, _P), np.float32)
_RT_NP[_ROWS, np.arange(_P)] += 1.0
_RT_NP[_COLS, np.arange(_P)] += 1.0


def _logic_kernel(x_ref, w_ref, b_ref, rt_ref, out_ref, w2_ref):
    wa = w_ref[0:_P, :]
    wo = w_ref[_P:2 * _P, :]
    wx = w_ref[2 * _P:3 * _P, :]
    a = wa - wo - 2.0 * wx
    c = wo + wx

    w2_ref[:, :] = jnp.zeros((_N * _N, _M), jnp.float32)
    for i in range(_N - 1):
        cnt = _N - 1 - i
        w2_ref[i * _N + i + 1:i * _N + _N, :] = a[int(_OFF[i]):int(_OFF[i]) + cnt, :]

    clin = jnp.dot(rt_ref[:, :], c, preferred_element_type=jnp.float32)

    x = x_ref[:, :]
    acc0 = jnp.dot(x, clin, preferred_element_type=jnp.float32) + b_ref[:, :]
    acc1 = jnp.zeros((_R, _M), jnp.float32)
    acc2 = jnp.zeros((_R, _M), jnp.float32)
    acc3 = jnp.zeros((_R, _M), jnp.float32)
    accs = [acc0, acc1, acc2, acc3]
    for g in range(_N // 4):
        lhs = jnp.concatenate(
            [x * x[:, 4 * g + u:4 * g + u + 1] for u in range(4)], axis=1)
        wblk = w2_ref[256 * g:256 * (g + 1), :]
        accs[g % 4] = accs[g % 4] + jnp.dot(
            lhs, wblk, preferred_element_type=jnp.float32)
    out_ref[:, :] = (accs[0] + accs[1]) + (accs[2] + accs[3])


def kernel(inputs, W, b):
    x2d = inputs.reshape(_R, _N)
    b2d = b.reshape(1, _M)
    rt = jnp.asarray(_RT_NP)
    out = pl.pallas_call(
        _logic_kernel,
        out_shape=jax.ShapeDtypeStruct((_R, _M), jnp.float32),
        scratch_shapes=[pltpu.VMEM((_N * _N, _M), jnp.float32)],
    )(x2d, W, b2d, rt)
    return out.reshape(_B, _K, _M)
